# Initial kernel scaffold; baseline (speedup 1.0000x reference)
import jax
import jax.numpy as jnp
from jax.experimental import pallas as pl


def kernel(x, edge_index, Wl1, Wr1, att1, b1, gamma, beta, Wl2, Wr2, att2, b2):
    raise NotImplementedError("write your pallas kernel here")


# speedup vs baseline: 20.8619x; 20.8619x over previous
import jax
import jax.numpy as jnp
from jax import lax
from jax.experimental import pallas as pl
from jax.experimental.pallas import tpu as pltpu
from jax.experimental.pallas import tpu_sc as plsc

N = 10000
E = 320000
D = 128
HID = 64

NPAD = 10240
NC, NS = 2, 16
NW = NC * NS
C = 128
EPW = 10368
EPAD = EPW * NW
NCHUNKS = EPW // C
RPT = NPAD // NS
AW = 80

_MESH = plsc.VectorSubcoreMesh(
    core_axis_name="c", subcore_axis_name="s", num_cores=NC, num_subcores=NS
)
_SC_PARAMS = pltpu.CompilerParams(
    needs_layout_passes=False, use_tc_tiling_on_sc=False
)


def _worker_prologue(msgb, zw):
    def zrow(i, carry):
        for j in range(zw):
            msgb[i, 16 * j:16 * (j + 1)] = jnp.zeros((16,), jnp.float32)
        return carry

    lax.fori_loop(0, C, zrow, 0)


def _zero_acc_stripe(acc, msgb, s, width):
    del width
    row0 = s * RPT
    for t in range(RPT // C):
        pltpu.sync_copy(msgb, acc.at[pl.ds(row0 + t * C, C)])


def _writeback(acc, msgb, out_hbm, c, s):
    row0 = s * RPT
    for t in range(RPT // C):
        r = row0 + t * C
        pltpu.sync_copy(acc.at[pl.ds(r, C)], msgb)
        pltpu.sync_copy(msgb, out_hbm.at[c, pl.ds(r, C)])


def _head_pass_body(xl_hbm, xr_hbm, att_hbm, src_hbm, dst_hbm, out_hbm,
                    srcv, dstv, xlb, xrb, msgb, attv, acc, sem1, sem2):
    c = lax.axis_index("c")
    s = lax.axis_index("s")
    wid = s * NC + c
    pltpu.sync_copy(att_hbm, attv)
    _worker_prologue(msgb, AW // 16)
    _zero_acc_stripe(acc, msgb, s, AW)
    plsc.subcore_barrier()

    ebase = wid * EPW

    def chunk(ci, carry):
        base = ebase + ci * C
        pltpu.sync_copy(src_hbm.at[pl.ds(base, C)], srcv)
        pltpu.sync_copy(dst_hbm.at[pl.ds(base, C)], dstv)
        g1 = pltpu.async_copy(xl_hbm.at[srcv], xlb, sem1)
        g2 = pltpu.async_copy(xr_hbm.at[dstv], xrb, sem2)
        g1.wait()
        g2.wait()

        def edge(e, ecarry):
            lane = lax.broadcasted_iota(jnp.int32, (16,), 0)
            xlv = [xlb[e, 16 * k:16 * (k + 1)] for k in range(4)]
            t = None
            for k in range(4):
                ek = xlv[k] + xrb[e, 16 * k:16 * (k + 1)]
                ek = jnp.maximum(ek, 0.2 * ek)
                pk = ek * attv[16 * k:16 * (k + 1)]
                t = pk if t is None else t + pk
            ev = jnp.exp(jnp.broadcast_to(jnp.sum(t), (16,)))
            for k in range(4):
                msgb[e, 16 * k:16 * (k + 1)] = xlv[k] * ev
            msgb[e, 64:80] = jnp.where(lane == 0, ev, 0.0)
            return ecarry

        lax.fori_loop(0, C, edge, 0)
        pltpu.sync_copy(msgb, acc.at[dstv], add=True)
        return carry

    lax.fori_loop(0, NCHUNKS, chunk, 0)
    plsc.subcore_barrier()
    _writeback(acc, msgb, out_hbm, c, s)


_head_pass = pl.kernel(
    _head_pass_body,
    out_type=jax.ShapeDtypeStruct((NC, NPAD, AW), jnp.float32),
    mesh=_MESH,
    compiler_params=_SC_PARAMS,
    scratch_types=[
        pltpu.VMEM((C,), jnp.int32),
        pltpu.VMEM((C,), jnp.int32),
        pltpu.VMEM((C, HID), jnp.float32),
        pltpu.VMEM((C, HID), jnp.float32),
        pltpu.VMEM((C, AW), jnp.float32),
        pltpu.VMEM((HID,), jnp.float32),
        pltpu.VMEM_SHARED((NPAD, AW), jnp.float32),
        pltpu.SemaphoreType.DMA,
        pltpu.SemaphoreType.DMA,
    ],
)


def _l2a_body(xl_hbm, xr_hbm, att_hbm, src_hbm, dst_hbm, out_hbm, ex_hbm,
              srcv, dstv, xlb, xrb, msgb, exv, attv, acc, sem1, sem2):
    c = lax.axis_index("c")
    s = lax.axis_index("s")
    wid = s * NC + c
    pltpu.sync_copy(att_hbm, attv)
    _worker_prologue(msgb, AW // 16)
    _zero_acc_stripe(acc, msgb, s, AW)
    plsc.subcore_barrier()

    ebase = wid * EPW

    def chunk(ci, carry):
        base = ebase + ci * C
        pltpu.sync_copy(src_hbm.at[pl.ds(base, C)], srcv)
        pltpu.sync_copy(dst_hbm.at[pl.ds(base, C)], dstv)
        g1 = pltpu.async_copy(xl_hbm.at[srcv], xlb, sem1)
        g2 = pltpu.async_copy(xr_hbm.at[dstv], xrb, sem2)
        g1.wait()
        g2.wait()

        def edge(e, exacc):
            lane = lax.broadcasted_iota(jnp.int32, (16,), 0)
            xlv = [xlb[e, 16 * k:16 * (k + 1)] for k in range(8)]
            t = None
            for k in range(8):
                ek = xlv[k] + xrb[e, 16 * k:16 * (k + 1)]
                ek = jnp.maximum(ek, 0.2 * ek)
                pk = ek * attv[16 * k:16 * (k + 1)]
                t = pk if t is None else t + pk
            ev = jnp.exp(jnp.broadcast_to(jnp.sum(t), (16,)))
            for k in range(4):
                msgb[e, 16 * k:16 * (k + 1)] = xlv[k] * ev
            msgb[e, 64:80] = jnp.where(lane == 0, ev, 0.0)
            exacc = jnp.where(lane == (e & 15), ev, exacc)

            @pl.when((e & 15) == 15)
            def _():
                exv[pl.ds((e >> 4) * 16, 16)] = exacc

            return exacc

        lax.fori_loop(0, C, edge, jnp.zeros((16,), jnp.float32))
        pltpu.sync_copy(msgb, acc.at[dstv], add=True)
        pltpu.sync_copy(exv, ex_hbm.at[pl.ds(base, C)])
        return carry

    lax.fori_loop(0, NCHUNKS, chunk, 0)
    plsc.subcore_barrier()
    _writeback(acc, msgb, out_hbm, c, s)


_l2a_pass = pl.kernel(
    _l2a_body,
    out_type=(
        jax.ShapeDtypeStruct((NC, NPAD, AW), jnp.float32),
        jax.ShapeDtypeStruct((EPAD,), jnp.float32),
    ),
    mesh=_MESH,
    compiler_params=_SC_PARAMS,
    scratch_types=[
        pltpu.VMEM((C,), jnp.int32),
        pltpu.VMEM((C,), jnp.int32),
        pltpu.VMEM((C, D), jnp.float32),
        pltpu.VMEM((C, D), jnp.float32),
        pltpu.VMEM((C, AW), jnp.float32),
        pltpu.VMEM((C,), jnp.float32),
        pltpu.VMEM((D,), jnp.float32),
        pltpu.VMEM_SHARED((NPAD, AW), jnp.float32),
        pltpu.SemaphoreType.DMA,
        pltpu.SemaphoreType.DMA,
    ],
)


def _l2b_body(xh_hbm, ex_hbm, src_hbm, dst_hbm, out_hbm,
              srcv, dstv, xlb, msgb, exv, acc, sem1):
    c = lax.axis_index("c")
    s = lax.axis_index("s")
    wid = s * NC + c
    _worker_prologue(msgb, HID // 16)
    _zero_acc_stripe(acc, msgb, s, HID)
    plsc.subcore_barrier()

    ebase = wid * EPW

    def chunk(ci, carry):
        base = ebase + ci * C
        pltpu.sync_copy(src_hbm.at[pl.ds(base, C)], srcv)
        pltpu.sync_copy(dst_hbm.at[pl.ds(base, C)], dstv)
        pltpu.sync_copy(ex_hbm.at[pl.ds(base, C)], exv.at[pl.ds(0, C)])
        g1 = pltpu.async_copy(xh_hbm.at[srcv], xlb, sem1)
        g1.wait()

        def edge(e, ecarry):
            ev = jnp.broadcast_to(exv[pl.ds(e, 16)][0], (16,))
            for k in range(4):
                msgb[e, 16 * k:16 * (k + 1)] = (
                    xlb[e, 16 * k:16 * (k + 1)] * ev)
            return ecarry

        lax.fori_loop(0, C, edge, 0)
        pltpu.sync_copy(msgb, acc.at[dstv], add=True)
        return carry

    lax.fori_loop(0, NCHUNKS, chunk, 0)
    plsc.subcore_barrier()
    _writeback(acc, msgb, out_hbm, c, s)


_l2b_pass = pl.kernel(
    _l2b_body,
    out_type=jax.ShapeDtypeStruct((NC, NPAD, HID), jnp.float32),
    mesh=_MESH,
    compiler_params=_SC_PARAMS,
    scratch_types=[
        pltpu.VMEM((C,), jnp.int32),
        pltpu.VMEM((C,), jnp.int32),
        pltpu.VMEM((C, HID), jnp.float32),
        pltpu.VMEM((C, HID), jnp.float32),
        pltpu.VMEM((C + 16,), jnp.float32),
        pltpu.VMEM_SHARED((NPAD, HID), jnp.float32),
        pltpu.SemaphoreType.DMA,
    ],
)

BR = 256


def _mm_pair(xp, Wl, Wr):
    def body(x_ref, wl_ref, wr_ref, a_ref, b_ref, c_ref, d_ref):
        xb = x_ref[...]
        xl = jnp.dot(xb, wl_ref[...], preferred_element_type=jnp.float32)
        xr = jnp.dot(xb, wr_ref[...], preferred_element_type=jnp.float32)
        a_ref[...] = xl[:, :HID]
        b_ref[...] = xl[:, HID:]
        c_ref[...] = xr[:, :HID]
        d_ref[...] = xr[:, HID:]

    return pl.pallas_call(
        body,
        grid=(NPAD // BR,),
        in_specs=[
            pl.BlockSpec((BR, D), lambda i: (i, 0)),
            pl.BlockSpec((D, D), lambda i: (0, 0)),
            pl.BlockSpec((D, D), lambda i: (0, 0)),
        ],
        out_specs=[pl.BlockSpec((BR, HID), lambda i: (i, 0))] * 4,
        out_shape=[jax.ShapeDtypeStruct((NPAD, HID), jnp.float32)] * 4,
    )(xp, Wl, Wr)


def _mid(p0, p1, b1, gamma, beta, Wl2, Wr2):
    def body(a0_ref, a1_ref, b0_ref, b1_ref, bias_ref, g_ref, be_ref,
             wl_ref, wr_ref, o1_ref, o2_ref, o3_ref):
        a = a0_ref[0] + a1_ref[0]
        b = b0_ref[0] + b1_ref[0]
        den0 = jnp.broadcast_to(a[:, HID:HID + 1], (BR, HID))
        den1 = jnp.broadcast_to(b[:, HID:HID + 1], (BR, HID))
        h = jnp.concatenate(
            [a[:, :HID] / (den0 + 1e-16), b[:, :HID] / (den1 + 1e-16)],
            axis=1) + bias_ref[...]
        mu = jnp.mean(h, axis=1, keepdims=True)
        var = jnp.mean((h - mu) ** 2, axis=1, keepdims=True)
        hn = (h - mu) / jnp.sqrt(var + 1e-5) * g_ref[...] + be_ref[...]
        he = jnp.where(hn > 0, hn, jnp.exp(hn) - 1.0)
        xl2 = jnp.dot(he, wl_ref[...], preferred_element_type=jnp.float32)
        o1_ref[...] = xl2
        o2_ref[...] = jnp.dot(he, wr_ref[...], preferred_element_type=jnp.float32)
        o3_ref[...] = xl2[:, HID:]

    return pl.pallas_call(
        body,
        grid=(NPAD // BR,),
        in_specs=[
            pl.BlockSpec((1, BR, AW), lambda i: (0, i, 0)),
            pl.BlockSpec((1, BR, AW), lambda i: (1, i, 0)),
            pl.BlockSpec((1, BR, AW), lambda i: (0, i, 0)),
            pl.BlockSpec((1, BR, AW), lambda i: (1, i, 0)),
            pl.BlockSpec((1, D), lambda i: (0, 0)),
            pl.BlockSpec((1, D), lambda i: (0, 0)),
            pl.BlockSpec((1, D), lambda i: (0, 0)),
            pl.BlockSpec((D, D), lambda i: (0, 0)),
            pl.BlockSpec((D, D), lambda i: (0, 0)),
        ],
        out_specs=[
            pl.BlockSpec((BR, D), lambda i: (i, 0)),
            pl.BlockSpec((BR, D), lambda i: (i, 0)),
            pl.BlockSpec((BR, HID), lambda i: (i, 0)),
        ],
        out_shape=[
            jax.ShapeDtypeStruct((NPAD, D), jnp.float32),
            jax.ShapeDtypeStruct((NPAD, D), jnp.float32),
            jax.ShapeDtypeStruct((NPAD, HID), jnp.float32),
        ],
    )(p0, p0, p1, p1, b1, gamma, beta, Wl2, Wr2)


BF = 400


def _final(pa, pb, b2):
    def body(a0_ref, a1_ref, c0_ref, c1_ref, b_ref, node_ref, g_ref):
        i = pl.program_id(0)
        a = a0_ref[0] + a1_ref[0]
        hi = c0_ref[0] + c1_ref[0]
        den = jnp.broadcast_to(a[:, HID:HID + 1], (BF, HID))
        node = jnp.concatenate(
            [a[:, :HID] / (den + 1e-16), hi / (den + 1e-16)],
            axis=1) + b_ref[...]
        node_ref[...] = node

        @pl.when(i == 0)
        def _():
            g_ref[...] = jnp.zeros_like(g_ref)

        g_ref[...] += jnp.sum(node, axis=0, keepdims=True) * (1.0 / N)

    return pl.pallas_call(
        body,
        grid=(N // BF,),
        in_specs=[
            pl.BlockSpec((1, BF, AW), lambda i: (0, i, 0)),
            pl.BlockSpec((1, BF, AW), lambda i: (1, i, 0)),
            pl.BlockSpec((1, BF, HID), lambda i: (0, i, 0)),
            pl.BlockSpec((1, BF, HID), lambda i: (1, i, 0)),
            pl.BlockSpec((1, D), lambda i: (0, 0)),
        ],
        out_specs=[
            pl.BlockSpec((BF, D), lambda i: (i, 0)),
            pl.BlockSpec((1, D), lambda i: (0, 0)),
        ],
        out_shape=[
            jax.ShapeDtypeStruct((N, D), jnp.float32),
            jax.ShapeDtypeStruct((1, D), jnp.float32),
        ],
    )(pa, pa, pb, pb, b2)


def kernel(x, edge_index, Wl1, Wr1, att1, b1, gamma, beta, Wl2, Wr2, att2, b2):
    xp = jnp.zeros((NPAD, D), jnp.float32).at[:N].set(x)
    sl = jnp.arange(N, dtype=jnp.int32)
    pad = jnp.full((EPAD - E - N,), N, dtype=jnp.int32)
    src = jnp.concatenate([edge_index[0], sl, pad])
    dst = jnp.concatenate([edge_index[1], sl, pad])
    att1f = att1.reshape(-1)

    xl_h0, xl_h1, xr_h0, xr_h1 = _mm_pair(xp, Wl1, Wr1)
    p_h0 = _head_pass(xl_h0, xr_h0, att1f[:HID], src, dst)
    p_h1 = _head_pass(xl_h1, xr_h1, att1f[HID:], src, dst)
    xl2, xr2, xl2hi = _mid(p_h0, p_h1, b1.reshape(1, D), gamma.reshape(1, D),
                           beta.reshape(1, D), Wl2, Wr2)
    pa, exb = _l2a_pass(xl2, xr2, att2.reshape(-1), src, dst)
    pb = _l2b_pass(xl2hi, exb, src, dst)
    node_emb, graph_emb = _final(pa, pb, b2.reshape(1, D))
    return node_emb, graph_emb


# speedup vs baseline: 24.7285x; 1.1853x over previous
import jax
import jax.numpy as jnp
from jax import lax
from jax.experimental import pallas as pl
from jax.experimental.pallas import tpu as pltpu
from jax.experimental.pallas import tpu_sc as plsc

N = 10000
E = 320000
D = 128
HID = 64

NPAD = 10240
NC, NS = 2, 16
NW = NC * NS
C = 128
NCHUNKS = 82
NPAIR = NCHUNKS // 2
EPW = NCHUNKS * C
EPAD = EPW * NW
RPT = NPAD // NS
AW = 80

_MESH = plsc.VectorSubcoreMesh(
    core_axis_name="c", subcore_axis_name="s", num_cores=NC, num_subcores=NS
)
_SC_PARAMS = pltpu.CompilerParams(
    needs_layout_passes=False, use_tc_tiling_on_sc=False
)


def _make_pass(mode):
    gw = D if mode == "l2a" else HID
    ow = HID if mode == "l2b" else AW
    gk = gw // 16
    cc = 64 if mode == "l2a" else C
    nch = EPW // cc
    npair = nch // 2

    def run(xl_hbm, xr_hbm, att_hbm, ex_hbm, src_hbm, dst_hbm, out_hbm,
            srcv, dstv, xlb, xrb, msgb, exv, attv, acc, semxl, semxr):
        c = lax.axis_index("c")
        s = lax.axis_index("s")
        wid = s * NC + c
        ebase = wid * EPW
        if mode != "l2b":
            pltpu.sync_copy(att_hbm, attv)

        def zrow(i, carry):
            for j in range(ow // 16):
                msgb[0, i, 16 * j:16 * (j + 1)] = jnp.zeros((16,), jnp.float32)
            return carry

        lax.fori_loop(0, cc, zrow, 0)
        row0 = s * RPT
        for t in range(RPT // cc):
            pltpu.sync_copy(msgb.at[0], acc.at[pl.ds(row0 + t * cc, cc)])
        plsc.subcore_barrier()

        lane = lax.broadcasted_iota(jnp.int32, (16,), 0)
        att_regs = ([attv[16 * k:16 * (k + 1)] for k in range(gk)]
                    if mode != "l2b" else None)

        def start(ci, b):
            base = ebase + ci * cc
            pltpu.sync_copy(src_hbm.at[pl.ds(base, cc)], srcv.at[b])
            pltpu.sync_copy(dst_hbm.at[pl.ds(base, cc)], dstv.at[b])
            cp1 = pltpu.async_copy(
                xl_hbm.at[srcv.at[b]], xlb.at[b], semxl.at[b])
            del cp1
            if mode != "l2b":
                cp2 = pltpu.async_copy(
                    xr_hbm.at[dstv.at[b]], xrb.at[b], semxr.at[b])
                del cp2

        def wait(b):
            pltpu.make_async_copy(xl_hbm.at[srcv.at[b]],
                                  xlb.at[b], semxl.at[b]).wait()
            if mode != "l2b":
                pltpu.make_async_copy(xr_hbm.at[dstv.at[b]],
                                      xrb.at[b], semxr.at[b]).wait()

        def process(ci, b):
            base = ebase + ci * cc
            if mode == "l2b":
                pltpu.sync_copy(ex_hbm.at[pl.ds(base, cc)],
                                exv.at[b, pl.ds(0, cc)])
            wait(b)

            if mode == "l2b":
                def edge(e, ecarry):
                    ev = jnp.broadcast_to(exv[b, pl.ds(e, 16)][0], (16,))
                    for k in range(4):
                        msgb[b, e, 16 * k:16 * (k + 1)] = (
                            xlb[b, e, 16 * k:16 * (k + 1)] * ev)
                    return ecarry

                lax.fori_loop(0, cc, edge, 0, unroll=2)
            else:
                def edge(e, exacc):
                    xlv = [xlb[b, e, 16 * k:16 * (k + 1)] for k in range(gk)]
                    t = None
                    for k in range(gk):
                        ek = xlv[k] + xrb[b, e, 16 * k:16 * (k + 1)]
                        ek = jnp.maximum(ek, 0.2 * ek)
                        pk = ek * att_regs[k]
                        t = pk if t is None else t + pk
                    ev = jnp.exp(jnp.broadcast_to(jnp.sum(t), (16,)))
                    for k in range(4):
                        msgb[b, e, 16 * k:16 * (k + 1)] = xlv[k] * ev
                    msgb[b, e, 64:80] = jnp.where(lane == 0, ev, 0.0)
                    if mode == "l2a":
                        exacc = jnp.where(lane == (e & 15), ev, exacc)

                        @pl.when((e & 15) == 15)
                        def _():
                            exv[b, pl.ds((e >> 4) * 16, 16)] = exacc

                    return exacc

                lax.fori_loop(0, cc, edge, jnp.zeros((16,), jnp.float32),
                              unroll=2)
            pltpu.sync_copy(msgb.at[b], acc.at[dstv.at[b]], add=True)
            if mode == "l2a":
                pltpu.sync_copy(exv.at[b, pl.ds(0, cc)],
                                ex_hbm.at[pl.ds(base, cc)])

        start(0, 0)

        def pair(j, carry):
            start(2 * j + 1, 1)
            process(2 * j, 0)

            @pl.when(j < npair - 1)
            def _():
                start(2 * j + 2, 0)

            process(2 * j + 1, 1)
            return carry

        lax.fori_loop(0, npair, pair, 0)
        plsc.subcore_barrier()

        for t in range(RPT // cc):
            r = row0 + t * cc
            pltpu.sync_copy(acc.at[pl.ds(r, cc)], msgb.at[0])
            pltpu.sync_copy(msgb.at[0], out_hbm.at[c, pl.ds(r, cc)])

    if mode == "head":
        def body(xl, xr, att, src, dst, out,
                 srcv, dstv, xlb, xrb, msgb, attv, acc, semxl, semxr):
            run(xl, xr, att, None, src, dst, out,
                srcv, dstv, xlb, xrb, msgb, None, attv, acc, semxl, semxr)

        out_type = jax.ShapeDtypeStruct((NC, NPAD, AW), jnp.float32)
        scratch = [
            pltpu.VMEM((2, C), jnp.int32),
            pltpu.VMEM((2, C), jnp.int32),
            pltpu.VMEM((2, C, HID), jnp.float32),
            pltpu.VMEM((2, C, HID), jnp.float32),
            pltpu.VMEM((2, C, AW), jnp.float32),
            pltpu.VMEM((HID,), jnp.float32),
            pltpu.VMEM_SHARED((NPAD, AW), jnp.float32),
            pltpu.SemaphoreType.DMA((2,)),
            pltpu.SemaphoreType.DMA((2,)),
        ]
    elif mode == "l2a":
        def body(xl, xr, att, src, dst, out, ex,
                 srcv, dstv, xlb, xrb, msgb, exv, attv, acc,
                 semxl, semxr):
            run(xl, xr, att, ex, src, dst, out,
                srcv, dstv, xlb, xrb, msgb, exv, attv, acc, semxl, semxr)

        out_type = (
            jax.ShapeDtypeStruct((NC, NPAD, AW), jnp.float32),
            jax.ShapeDtypeStruct((EPAD,), jnp.float32),
        )
        scratch = [
            pltpu.VMEM((2, 64), jnp.int32),
            pltpu.VMEM((2, 64), jnp.int32),
            pltpu.VMEM((2, 64, D), jnp.float32),
            pltpu.VMEM((2, 64, D), jnp.float32),
            pltpu.VMEM((2, 64, AW), jnp.float32),
            pltpu.VMEM((2, 64), jnp.float32),
            pltpu.VMEM((D,), jnp.float32),
            pltpu.VMEM_SHARED((NPAD, AW), jnp.float32),
            pltpu.SemaphoreType.DMA((2,)),
            pltpu.SemaphoreType.DMA((2,)),
        ]
    else:
        def body(xh, ex, src, dst, out,
                 srcv, dstv, xlb, msgb, exv, acc, semxl):
            run(xh, None, None, ex, src, dst, out,
                srcv, dstv, xlb, None, msgb, exv, None, acc, semxl, None)

        out_type = jax.ShapeDtypeStruct((NC, NPAD, HID), jnp.float32)
        scratch = [
            pltpu.VMEM((2, C), jnp.int32),
            pltpu.VMEM((2, C), jnp.int32),
            pltpu.VMEM((2, C, HID), jnp.float32),
            pltpu.VMEM((2, C, HID), jnp.float32),
            pltpu.VMEM((2, C + 16), jnp.float32),
            pltpu.VMEM_SHARED((NPAD, HID), jnp.float32),
            pltpu.SemaphoreType.DMA((2,)),
        ]

    return pl.kernel(
        body,
        out_type=out_type,
        mesh=_MESH,
        compiler_params=_SC_PARAMS,
        scratch_types=scratch,
    )


_head_pass = _make_pass("head")
_l2a_pass = _make_pass("l2a")
_l2b_pass = _make_pass("l2b")

BR = 256


def _mm_pair(xp, Wl, Wr):
    def body(x_ref, wl_ref, wr_ref, a_ref, b_ref, c_ref, d_ref):
        xb = x_ref[...]
        xl = jnp.dot(xb, wl_ref[...], preferred_element_type=jnp.float32)
        xr = jnp.dot(xb, wr_ref[...], preferred_element_type=jnp.float32)
        a_ref[...] = xl[:, :HID]
        b_ref[...] = xl[:, HID:]
        c_ref[...] = xr[:, :HID]
        d_ref[...] = xr[:, HID:]

    return pl.pallas_call(
        body,
        grid=(NPAD // BR,),
        in_specs=[
            pl.BlockSpec((BR, D), lambda i: (i, 0)),
            pl.BlockSpec((D, D), lambda i: (0, 0)),
            pl.BlockSpec((D, D), lambda i: (0, 0)),
        ],
        out_specs=[pl.BlockSpec((BR, HID), lambda i: (i, 0))] * 4,
        out_shape=[jax.ShapeDtypeStruct((NPAD, HID), jnp.float32)] * 4,
    )(xp, Wl, Wr)


def _mid(p0, p1, b1, gamma, beta, Wl2, Wr2):
    def body(a0_ref, a1_ref, b0_ref, b1_ref, bias_ref, g_ref, be_ref,
             wl_ref, wr_ref, o1_ref, o2_ref, o3_ref):
        a = a0_ref[0] + a1_ref[0]
        b = b0_ref[0] + b1_ref[0]
        den0 = jnp.broadcast_to(a[:, HID:HID + 1], (BR, HID))
        den1 = jnp.broadcast_to(b[:, HID:HID + 1], (BR, HID))
        h = jnp.concatenate(
            [a[:, :HID] / (den0 + 1e-16), b[:, :HID] / (den1 + 1e-16)],
            axis=1) + bias_ref[...]
        mu = jnp.mean(h, axis=1, keepdims=True)
        var = jnp.mean((h - mu) ** 2, axis=1, keepdims=True)
        hn = (h - mu) / jnp.sqrt(var + 1e-5) * g_ref[...] + be_ref[...]
        he = jnp.where(hn > 0, hn, jnp.exp(hn) - 1.0)
        xl2 = jnp.dot(he, wl_ref[...], preferred_element_type=jnp.float32)
        o1_ref[...] = xl2
        o2_ref[...] = jnp.dot(he, wr_ref[...], preferred_element_type=jnp.float32)
        o3_ref[...] = xl2[:, HID:]

    return pl.pallas_call(
        body,
        grid=(NPAD // BR,),
        in_specs=[
            pl.BlockSpec((1, BR, AW), lambda i: (0, i, 0)),
            pl.BlockSpec((1, BR, AW), lambda i: (1, i, 0)),
            pl.BlockSpec((1, BR, AW), lambda i: (0, i, 0)),
            pl.BlockSpec((1, BR, AW), lambda i: (1, i, 0)),
            pl.BlockSpec((1, D), lambda i: (0, 0)),
            pl.BlockSpec((1, D), lambda i: (0, 0)),
            pl.BlockSpec((1, D), lambda i: (0, 0)),
            pl.BlockSpec((D, D), lambda i: (0, 0)),
            pl.BlockSpec((D, D), lambda i: (0, 0)),
        ],
        out_specs=[
            pl.BlockSpec((BR, D), lambda i: (i, 0)),
            pl.BlockSpec((BR, D), lambda i: (i, 0)),
            pl.BlockSpec((BR, HID), lambda i: (i, 0)),
        ],
        out_shape=[
            jax.ShapeDtypeStruct((NPAD, D), jnp.float32),
            jax.ShapeDtypeStruct((NPAD, D), jnp.float32),
            jax.ShapeDtypeStruct((NPAD, HID), jnp.float32),
        ],
    )(p0, p0, p1, p1, b1, gamma, beta, Wl2, Wr2)


BF = 400


def _final(pa, pb, b2):
    def body(a0_ref, a1_ref, c0_ref, c1_ref, b_ref, node_ref, g_ref):
        i = pl.program_id(0)
        a = a0_ref[0] + a1_ref[0]
        hi = c0_ref[0] + c1_ref[0]
        den = jnp.broadcast_to(a[:, HID:HID + 1], (BF, HID))
        node = jnp.concatenate(
            [a[:, :HID] / (den + 1e-16), hi / (den + 1e-16)],
            axis=1) + b_ref[...]
        node_ref[...] = node

        @pl.when(i == 0)
        def _():
            g_ref[...] = jnp.zeros_like(g_ref)

        g_ref[...] += jnp.sum(node, axis=0, keepdims=True) * (1.0 / N)

    return pl.pallas_call(
        body,
        grid=(N // BF,),
        in_specs=[
            pl.BlockSpec((1, BF, AW), lambda i: (0, i, 0)),
            pl.BlockSpec((1, BF, AW), lambda i: (1, i, 0)),
            pl.BlockSpec((1, BF, HID), lambda i: (0, i, 0)),
            pl.BlockSpec((1, BF, HID), lambda i: (1, i, 0)),
            pl.BlockSpec((1, D), lambda i: (0, 0)),
        ],
        out_specs=[
            pl.BlockSpec((BF, D), lambda i: (i, 0)),
            pl.BlockSpec((1, D), lambda i: (0, 0)),
        ],
        out_shape=[
            jax.ShapeDtypeStruct((N, D), jnp.float32),
            jax.ShapeDtypeStruct((1, D), jnp.float32),
        ],
    )(pa, pa, pb, pb, b2)


def kernel(x, edge_index, Wl1, Wr1, att1, b1, gamma, beta, Wl2, Wr2, att2, b2):
    xp = jnp.zeros((NPAD, D), jnp.float32).at[:N].set(x)
    sl = jnp.arange(N, dtype=jnp.int32)
    pad = jnp.full((EPAD - E - N,), N, dtype=jnp.int32)
    src = jnp.concatenate([edge_index[0], sl, pad])
    dst = jnp.concatenate([edge_index[1], sl, pad])
    att1f = att1.reshape(-1)

    xl_h0, xl_h1, xr_h0, xr_h1 = _mm_pair(xp, Wl1, Wr1)
    p_h0 = _head_pass(xl_h0, xr_h0, att1f[:HID], src, dst)
    p_h1 = _head_pass(xl_h1, xr_h1, att1f[HID:], src, dst)
    xl2, xr2, xl2hi = _mid(p_h0, p_h1, b1.reshape(1, D), gamma.reshape(1, D),
                           beta.reshape(1, D), Wl2, Wr2)
    pa, exb = _l2a_pass(xl2, xr2, att2.reshape(-1), src, dst)
    pb = _l2b_pass(xl2hi, exb, src, dst)
    node_emb, graph_emb = _final(pa, pb, b2.reshape(1, D))
    return node_emb, graph_emb
